# Initial kernel scaffold; baseline (speedup 1.0000x reference)
import jax
import jax.numpy as jnp
from jax.experimental import pallas as pl


def kernel(x, table):
    raise NotImplementedError("write your pallas kernel here")


# speedup vs baseline: 1.0951x; 1.0951x over previous
import functools

import jax
import jax.numpy as jnp
from jax import lax
from jax.experimental import pallas as pl
from jax.experimental.pallas import tpu as pltpu
from jax.experimental.pallas import tpu_sc as plsc

NC = 2
NS = 16
NW = NC * NS

BATCH = 16384
HIST = 50
VECDIM = 32
B = BATCH * HIST
BPW = B // NW
CHUNK = 1024
NCHUNK = BPW // CHUNK

_mesh = plsc.VectorSubcoreMesh(
    core_axis_name="c", subcore_axis_name="s", num_cores=NC, num_subcores=NS
)


@functools.partial(
    pl.kernel,
    out_type=jax.ShapeDtypeStruct((B, VECDIM), jnp.float32),
    mesh=_mesh,
    scratch_types=[
        pltpu.VMEM((CHUNK,), jnp.int32),
        pltpu.VMEM((CHUNK, VECDIM), jnp.float32),
        pltpu.SemaphoreType.DMA,
    ],
    compiler_params=pltpu.CompilerParams(use_tc_tiling_on_sc=False),
)
def _gather_kernel(idx_hbm, table_hbm, out_hbm, idx_v, rows_v, sem):
    wid = lax.axis_index("s") * NC + lax.axis_index("c")
    base = wid * BPW

    @pl.loop(0, NCHUNK)
    def _chunk(i):
        off = base + i * CHUNK
        pltpu.sync_copy(idx_hbm.at[pl.ds(off, CHUNK)], idx_v)
        pltpu.async_copy(table_hbm.at[idx_v], rows_v, sem).wait()
        pltpu.sync_copy(rows_v, out_hbm.at[pl.ds(off, CHUNK)])


def kernel(x, table):
    idx = x.reshape(B).astype(jnp.int32)
    out = _gather_kernel(idx, table)
    return out.reshape(BATCH, HIST, VECDIM)


# speedup vs baseline: 1.1094x; 1.0131x over previous
import functools

import jax
import jax.numpy as jnp
from jax import lax
from jax.experimental import pallas as pl
from jax.experimental.pallas import tpu as pltpu
from jax.experimental.pallas import tpu_sc as plsc

NC = 2
NS = 16
NW = NC * NS

BATCH = 16384
HIST = 50
VECDIM = 32
B = BATCH * HIST
BPW = B // NW
CHUNK = 1600
NCHUNK = BPW // CHUNK

_mesh = plsc.VectorSubcoreMesh(
    core_axis_name="c", subcore_axis_name="s", num_cores=NC, num_subcores=NS
)


@functools.partial(
    pl.kernel,
    out_type=jax.ShapeDtypeStruct((B, VECDIM), jnp.float32),
    mesh=_mesh,
    scratch_types=[
        pltpu.VMEM((BPW,), jnp.int32),
        pltpu.VMEM((CHUNK, VECDIM), jnp.float32),
        pltpu.VMEM((CHUNK, VECDIM), jnp.float32),
        pltpu.SemaphoreType.DMA,
        pltpu.SemaphoreType.DMA,
        pltpu.SemaphoreType.DMA,
        pltpu.SemaphoreType.DMA,
    ],
    compiler_params=pltpu.CompilerParams(use_tc_tiling_on_sc=False),
)
def _gather_kernel(idx_hbm, table_hbm, out_hbm, idx_v, rows0, rows1,
                   gsem0, gsem1, wsem0, wsem1):
    wid = lax.axis_index("s") * NC + lax.axis_index("c")
    base = wid * BPW

    pltpu.sync_copy(idx_hbm.at[pl.ds(base, BPW)], idx_v)

    rows = (rows0, rows1)
    gsem = (gsem0, gsem1)
    wsem = (wsem0, wsem1)

    def start_gather(i):
        b = i % 2
        return pltpu.async_copy(
            table_hbm.at[idx_v.at[pl.ds(i * CHUNK, CHUNK)]], rows[b], gsem[b]
        )

    def start_write(i):
        b = i % 2
        return pltpu.async_copy(
            rows[b], out_hbm.at[pl.ds(base + i * CHUNK, CHUNK)], wsem[b]
        )

    g_desc = [None, None]
    w_desc = [None, None]
    g_desc[0] = start_gather(0)
    for i in range(1, NCHUNK):
        b = i % 2
        g_desc[1 - b].wait()
        w_desc[1 - b] = start_write(i - 1)
        if i >= 2:
            w_desc[b].wait()
        g_desc[b] = start_gather(i)
    last = NCHUNK - 1
    g_desc[last % 2].wait()
    w_desc[last % 2] = start_write(last)
    w_desc[(last - 1) % 2].wait()
    w_desc[last % 2].wait()


def kernel(x, table):
    idx = x.reshape(B).astype(jnp.int32)
    out = _gather_kernel(idx, table)
    return out.reshape(BATCH, HIST, VECDIM)


# speedup vs baseline: 1.5207x; 1.3707x over previous
import functools

import jax
import jax.numpy as jnp
from jax import lax
from jax.experimental import pallas as pl
from jax.experimental.pallas import tpu as pltpu
from jax.experimental.pallas import tpu_sc as plsc

NC = 2
NS = 16
NW = NC * NS

BATCH = 16384
HIST = 50
VECDIM = 32
B = BATCH * HIST
BPW = B // NW
BBLK = 128
NBT = BATCH // BBLK
BT_PER_W = NBT // NW
NBLOCK = BT_PER_W * HIST

_mesh = plsc.VectorSubcoreMesh(
    core_axis_name="c", subcore_axis_name="s", num_cores=NC, num_subcores=NS
)

_IOTA16 = tuple(range(16))


@functools.partial(
    pl.kernel,
    out_type=jax.ShapeDtypeStruct((HIST, VECDIM // 8, NBT, 8, BBLK), jnp.float32),
    mesh=_mesh,
    scratch_types=[
        pltpu.VMEM((BPW,), jnp.int32),
        pltpu.VMEM((BBLK,), jnp.int32),
        pltpu.VMEM((BBLK,), jnp.int32),
        pltpu.VMEM((BBLK, VECDIM), jnp.float32),
        pltpu.VMEM((BBLK, VECDIM), jnp.float32),
        pltpu.VMEM((VECDIM // 8, 8, BBLK), jnp.float32),
        pltpu.VMEM((VECDIM // 8, 8, BBLK), jnp.float32),
        pltpu.SemaphoreType.DMA,
        pltpu.SemaphoreType.DMA,
        pltpu.SemaphoreType.DMA,
        pltpu.SemaphoreType.DMA,
    ],
    compiler_params=pltpu.CompilerParams(
        use_tc_tiling_on_sc=False, needs_layout_passes=False
    ),
)
def _gather_kernel(idx_hbm, table_hbm, out_hbm, idx_v, ih0, ih1, r0, r1,
                   o0, o1, gsem0, gsem1, wsem0, wsem1):
    wid = lax.axis_index("s") * NC + lax.axis_index("c")
    base = wid * BPW

    ih = (ih0, ih1)
    rows = (r0, r1)
    oblk = (o0, o1)
    gsem = (gsem0, gsem1)
    wsem = (wsem0, wsem1)

    iota = jnp.arange(16, dtype=jnp.int32)

    pltpu.sync_copy(idx_hbm.at[pl.ds(base, BPW)], idx_v)

    def block_params(t):
        bt_i = t // HIST
        h = t - bt_i * HIST
        return bt_i, h

    def repack_and_gather(t, buf):
        bt_i, h = block_params(t)
        pos0 = bt_i * (HIST * BBLK) + h
        for k in range(8):
            vec = plsc.load_gather(idx_v, [iota * HIST + (pos0 + 800 * k)])
            ih[buf][pl.ds(16 * k, 16)] = vec
        return pltpu.async_copy(table_hbm.at[ih[buf]], rows[buf], gsem[buf])

    def wait_gather(buf):
        pltpu.make_async_copy(table_hbm.at[ih[buf]], rows[buf], gsem[buf]).wait()

    def transpose_and_write(t, buf):
        bt_i, h = block_params(t)
        for ct in range(VECDIM // 8):
            for s in range(8):
                c = jnp.full((16,), 8 * ct + s, dtype=jnp.int32)
                for m in range(8):
                    vec = plsc.load_gather(rows[buf], [iota + 16 * m, c])
                    oblk[buf][ct, s, pl.ds(16 * m, 16)] = vec
        dst = out_hbm.at[h, :, wid * BT_PER_W + bt_i]
        return pltpu.async_copy(oblk[buf], dst, wsem[buf])

    def wait_write(t, buf):
        bt_i, h = block_params(t)
        dst = out_hbm.at[h, :, wid * BT_PER_W + bt_i]
        pltpu.make_async_copy(oblk[buf], dst, wsem[buf]).wait()

    def half(t, tbuf):
        repack_and_gather(t, tbuf)
        prev = t - 1
        pbuf = 1 - tbuf
        wait_gather(pbuf)
        pl.when(t >= 3)(lambda: wait_write(prev - 2, pbuf))
        transpose_and_write(prev, pbuf)

    repack_and_gather(0, 0)

    @pl.loop(1, NBLOCK - 1, step=2)
    def _steady(t):
        half(t, 1)
        half(t + 1, 0)

    t_last = NBLOCK - 1
    repack_and_gather(t_last, 1)
    wait_gather(0)
    wait_write(t_last - 3, 0)
    transpose_and_write(t_last - 1, 0)
    wait_gather(1)
    wait_write(t_last - 2, 1)
    transpose_and_write(t_last, 1)
    wait_write(t_last - 1, 0)
    wait_write(t_last, 1)


def kernel(x, table):
    idx = x.reshape(B).astype(jnp.int32)
    out5d = _gather_kernel(idx, table)
    return out5d.transpose(2, 4, 0, 1, 3).reshape(BATCH, HIST, VECDIM)


# speedup vs baseline: 1.8943x; 1.2457x over previous
import functools

import jax
import jax.numpy as jnp
from jax import lax
from jax.experimental import pallas as pl
from jax.experimental.pallas import tpu as pltpu
from jax.experimental.pallas import tpu_sc as plsc

NC = 2
NS = 16
NW = NC * NS

BATCH = 16384
HIST = 50
VECDIM = 32
B = BATCH * HIST
BPW = B // NW
BBLK = 128
NBT = BATCH // BBLK
BT_PER_W = NBT // NW
NBLOCK = BT_PER_W * HIST

_mesh = plsc.VectorSubcoreMesh(
    core_axis_name="c", subcore_axis_name="s", num_cores=NC, num_subcores=NS
)

_IOTA16 = tuple(range(16))


@functools.partial(
    pl.kernel,
    out_type=jax.ShapeDtypeStruct((HIST, VECDIM // 8, NBT, 8, BBLK), jnp.float32),
    mesh=_mesh,
    scratch_types=[
        pltpu.VMEM((BPW,), jnp.int32),
        pltpu.VMEM((BBLK,), jnp.int32),
        pltpu.VMEM((BBLK,), jnp.int32),
        pltpu.VMEM((BBLK, VECDIM), jnp.float32),
        pltpu.VMEM((BBLK, VECDIM), jnp.float32),
        pltpu.VMEM((VECDIM // 8, 8, BBLK), jnp.float32),
        pltpu.VMEM((VECDIM // 8, 8, BBLK), jnp.float32),
        pltpu.SemaphoreType.DMA,
        pltpu.SemaphoreType.DMA,
        pltpu.SemaphoreType.DMA,
        pltpu.SemaphoreType.DMA,
    ],
    compiler_params=pltpu.CompilerParams(
        use_tc_tiling_on_sc=False,
        needs_layout_passes=False,
        disable_bounds_checks=True,
    ),
)
def _gather_kernel(idx_hbm, table_hbm, out_hbm, idx_v, ih0, ih1, r0, r1,
                   o0, o1, gsem0, gsem1, wsem0, wsem1):
    wid = lax.axis_index("s") * NC + lax.axis_index("c")
    base = wid * BPW

    ih = (ih0, ih1)
    rows = (r0, r1)
    oblk = (o0, o1)
    gsem = (gsem0, gsem1)
    wsem = (wsem0, wsem1)

    iota = jnp.arange(16, dtype=jnp.int32)

    pltpu.sync_copy(idx_hbm.at[pl.ds(base, BPW)], idx_v)

    def block_params(t):
        bt_i = t // HIST
        h = t - bt_i * HIST
        return bt_i, h

    def repack_and_gather(t, buf):
        bt_i, h = block_params(t)
        pos0 = bt_i * (HIST * BBLK) + h
        for k in range(8):
            vec = plsc.load_gather(idx_v, [iota * HIST + (pos0 + 800 * k)])
            ih[buf][pl.ds(16 * k, 16)] = vec
        return pltpu.async_copy(table_hbm.at[ih[buf]], rows[buf], gsem[buf])

    def wait_gather(buf):
        pltpu.make_async_copy(table_hbm.at[ih[buf]], rows[buf], gsem[buf]).wait()

    def transpose_and_write(t, buf):
        bt_i, h = block_params(t)

        @plsc.parallel_loop(0, VECDIM // 8, step=1, unroll=2)
        def _ct(ct):
            for s in range(8):
                c = jnp.full((16,), 8 * ct + s, dtype=jnp.int32)
                for m in range(8):
                    vec = plsc.load_gather(rows[buf], [iota + 16 * m, c])
                    oblk[buf][ct, s, pl.ds(16 * m, 16)] = vec

        dst = out_hbm.at[h, :, wid * BT_PER_W + bt_i]
        return pltpu.async_copy(oblk[buf], dst, wsem[buf])

    def wait_write(t, buf):
        bt_i, h = block_params(t)
        dst = out_hbm.at[h, :, wid * BT_PER_W + bt_i]
        pltpu.make_async_copy(oblk[buf], dst, wsem[buf]).wait()

    def half(t, tbuf):
        repack_and_gather(t, tbuf)
        prev = t - 1
        pbuf = 1 - tbuf
        wait_gather(pbuf)
        pl.when(t >= 3)(lambda: wait_write(prev - 2, pbuf))
        transpose_and_write(prev, pbuf)

    repack_and_gather(0, 0)

    @pl.loop(1, NBLOCK - 1, step=2)
    def _steady(t):
        half(t, 1)
        half(t + 1, 0)

    t_last = NBLOCK - 1
    repack_and_gather(t_last, 1)
    wait_gather(0)
    wait_write(t_last - 3, 0)
    transpose_and_write(t_last - 1, 0)
    wait_gather(1)
    wait_write(t_last - 2, 1)
    transpose_and_write(t_last, 1)
    wait_write(t_last - 1, 0)
    wait_write(t_last, 1)


def kernel(x, table):
    idx = x.reshape(B).astype(jnp.int32)
    out5d = _gather_kernel(idx, table)
    return out5d.transpose(2, 4, 0, 1, 3).reshape(BATCH, HIST, VECDIM)
